# baseline (speedup 1.0000x reference)
import jax
import jax.numpy as jnp
from jax import lax
from jax.experimental import pallas as pl
from jax.experimental.pallas import tpu as pltpu
from jax.experimental.pallas import tpu_sc as plsc

_CLS = 100000
_DIM = 32
_BATCH = 16384
_NC = 2
_NS = 16
_NW = _NC * _NS
_PER = _BATCH // _NW
_CHUNK = 128
_NHC = 8
_NGC = _PER // _CHUNK
_CNT_PAD = 100096
_ZCHUNK = _CNT_PAD // _NS // 2
_ACH = 512
_NF = 12
_TAILW = 256
_SROWS = _CLS // 4 + 24


def _body(ys_ref, xsT_ref, ct_ref, tail_ref, out_ref, scratch_ref,
          idx_v, idx4_v, xsT_v, in_v, out_v, c_v, cnt_v, ones_v, z_v, acc_v,
          cnt_sh, sem, sem_i, sem_h, sem_a, sem_o):
    c = lax.axis_index("c")
    s = lax.axis_index("s")
    wid = s * _NC + c
    lanes = lax.iota(jnp.int32, 16)
    zero16 = jnp.zeros((16,), jnp.float32)
    my_scr = scratch_ref.at[c]

    icopy = pltpu.async_copy(ys_ref.at[pl.ds(s * _NHC, _NHC)], idx_v, sem_i)
    xcopy = pltpu.async_copy(
        xsT_ref.at[:, pl.ds(wid * _PER, _PER)], xsT_v, sem)

    for k in range(_CHUNK // 16):
        ones_v[pl.ds(k * 16, 16)] = zero16 + 1.0

    def zstore(i, carry):
        z_v[pl.ds(i * 16, 16)] = zero16
        return carry

    lax.fori_loop(0, _ZCHUNK // 16, zstore, 0)
    pltpu.sync_copy(z_v, cnt_sh.at[pl.ds(s * 2 * _ZCHUNK, _ZCHUNK)])
    pltpu.sync_copy(z_v, cnt_sh.at[pl.ds((s * 2 + 1) * _ZCHUNK, _ZCHUNK)])
    plsc.subcore_barrier()

    icopy.wait()
    hist_copies = [
        pltpu.async_copy(ones_v, cnt_sh.at[idx_v.at[g]], sem_h, add=True)
        for g in range(_NHC)
    ]
    for g in range(_NGC):
        for k in range(_CHUNK // 16):
            y16 = idx_v[c * _NGC + g, pl.ds(k * 16, 16)]
            idx4_v[g, pl.ds(k * 16, 16)] = y16 >> 2

    dvec = [lanes + (q & 1) * 16 for q in range(8)]

    def transpose_chunk(buf, width):
        def trow(r, carry):
            cl0 = r * 4
            for q in range(8):
                clvec = jnp.full((16,), 1, jnp.int32) * (cl0 + (q >> 1))
                vals = plsc.load_gather(in_v.at[buf], [dvec[q], clvec])
                out_v[buf, r, pl.ds(q * 16, 16)] = vals
            return carry
        lax.fori_loop(0, width * _DIM // 128, trow, 0)

    def fire_in(k):
        cid = s + 16 * k
        return pltpu.async_copy(
            ct_ref.at[:, pl.ds(cid * _ACH, _ACH)], in_v.at[k % 2], sem_a)

    in_d = [fire_in(0), fire_in(1)]
    out_d = [None, None]
    for k in range(_NF):
        b = k % 2
        in_d[b].wait()
        if out_d[b] is not None:
            out_d[b].wait()
        transpose_chunk(b, _ACH)
        cid = s + 16 * k
        out_d[b] = pltpu.async_copy(
            out_v.at[b], my_scr.at[pl.ds(cid * (_ACH * _DIM // 128), 128)],
            sem_o)
        if k + 2 < _NF:
            in_d[b] = fire_in(k + 2)
    for d in out_d:
        d.wait()

    @pl.when(s < 3)
    def _full_epilogue():
        cid = s + 16 * _NF
        pltpu.sync_copy(ct_ref.at[:, pl.ds(cid * _ACH, _ACH)], in_v.at[0])
        transpose_chunk(0, _ACH)
        pltpu.sync_copy(out_v.at[0],
                        my_scr.at[pl.ds(cid * (_ACH * _DIM // 128), 128)])

    @pl.when(s == 3)
    def _tail_epilogue():
        pltpu.sync_copy(tail_ref, in_v.at[0].at[:, pl.ds(0, _TAILW)])
        transpose_chunk(0, _TAILW)
        pltpu.sync_copy(
            out_v.at[0].at[pl.ds(0, _TAILW * _DIM // 128)],
            my_scr.at[pl.ds(195 * (_ACH * _DIM // 128), _TAILW * _DIM // 128)])

    for h in hist_copies:
        h.wait()
    plsc.subcore_barrier()

    cnt_copies = [
        pltpu.async_copy(cnt_sh.at[idx_v.at[c * _NGC + g]],
                         cnt_v.at[pl.ds(g * _CHUNK, _CHUNK)], sem_h)
        for g in range(_NGC)
    ]
    xcopy.wait()

    def fire_c(g):
        return pltpu.async_copy(
            my_scr.at[idx4_v.at[g]], c_v.at[g % 2], sem)

    c_d = [fire_c(0), fire_c(1)]
    acc = zero16
    for g128 in range(_NGC):
        c_d[g128 % 2].wait()
        cnt_copies[g128].wait()

        def group(g, a):
            j0 = g128 * _CHUNK + g * 16
            rows = lanes + g * 16
            y16 = idx_v[c * _NGC + g128, pl.ds(g * 16, 16)]
            ccol0 = (y16 & 3) << 5
            cnt16 = plsc.load_gather(cnt_v, [lanes + j0])
            w16 = 0.5 / (cnt16 + 1.0)
            sq = zero16
            for d in range(_DIM):
                t = (xsT_v[d, pl.ds(j0, 16)]
                     - plsc.load_gather(c_v.at[g128 % 2], [rows, ccol0 + d]))
                sq = sq + t * t
            return a + sq * w16

        acc = lax.fori_loop(0, _CHUNK // 16, group, acc)
        if g128 + 2 < _NGC:
            c_d[g128 % 2] = fire_c(g128 + 2)
    acc_v[...] = acc
    pltpu.sync_copy(acc_v, out_ref.at[pl.ds(wid * 16, 16)])


def kernel(xs, ys, center):
    ys2d = ys.astype(jnp.int32).reshape(_NS * _NHC, _CHUNK)
    xsT = xs.T
    centerT = center.T
    tailT = jnp.pad(center[195 * _ACH:].T, ((0, 0), (0, _TAILW - 160)))
    mesh = plsc.VectorSubcoreMesh(
        core_axis_name="c", subcore_axis_name="s", num_cores=_NC)
    out, _ = pl.kernel(
        _body,
        out_type=(jax.ShapeDtypeStruct((_NW * 16,), jnp.float32),
                  jax.ShapeDtypeStruct((_NC, _SROWS, 128), jnp.float32)),
        mesh=mesh,
        compiler_params=pltpu.CompilerParams(
            needs_layout_passes=False, use_tc_tiling_on_sc=True),
        scratch_types=[
            pltpu.VMEM((_NHC, _CHUNK), jnp.int32),
            pltpu.VMEM((_NGC, _CHUNK), jnp.int32),
            pltpu.VMEM((_DIM, _PER), jnp.float32),
            pltpu.VMEM((2, _DIM, _ACH), jnp.float32),
            pltpu.VMEM((2, 128, 128), jnp.float32),
            pltpu.VMEM((2, _CHUNK, 128), jnp.float32),
            pltpu.VMEM((_PER,), jnp.float32),
            pltpu.VMEM((_CHUNK,), jnp.float32),
            pltpu.VMEM((_ZCHUNK,), jnp.float32),
            pltpu.VMEM((16,), jnp.float32),
            pltpu.VMEM_SHARED((_CNT_PAD,), jnp.float32),
            pltpu.SemaphoreType.DMA,
            pltpu.SemaphoreType.DMA,
            pltpu.SemaphoreType.DMA,
            pltpu.SemaphoreType.DMA,
            pltpu.SemaphoreType.DMA,
        ],
    )(ys2d, xsT, centerT, tailT)
    return jnp.sum(out)


# speedup vs baseline: 2.0600x; 2.0600x over previous
import jax
import jax.numpy as jnp
from jax import lax
from jax.experimental import pallas as pl
from jax.experimental.pallas import tpu as pltpu
from jax.experimental.pallas import tpu_sc as plsc

_CLS = 100000
_DIM = 32
_BATCH = 16384
_NC = 2
_NS = 16
_NW = _NC * _NS
_PER = _BATCH // _NW
_CHUNK = 128
_NHC = 8
_NGC = _PER // _CHUNK
_CNT_PAD = 100096
_ZCHUNK = _CNT_PAD // _NS // 2
_ACH = 512
_NF = 12
_TAILW = 256
_SROWS = _CLS // 4 + 24


def _body(ys_ref, xsT_ref, ct_ref, tail_ref, out_ref, scratch_ref,
          idx_v, idx4_v, xsT_v, in_v, out_v, c_v, cnt_v, ones_v, z_v, acc_v,
          cnt_sh, sem, sem_i, sem_h, sem_a, sem_o):
    c = lax.axis_index("c")
    s = lax.axis_index("s")
    wid = s * _NC + c
    lanes = lax.iota(jnp.int32, 16)
    zero16 = jnp.zeros((16,), jnp.float32)
    my_scr = scratch_ref.at[c]

    icopy = pltpu.async_copy(ys_ref.at[pl.ds(s * _NHC, _NHC)], idx_v, sem_i)
    xcopy = pltpu.async_copy(
        xsT_ref.at[:, pl.ds(wid * _PER, _PER)], xsT_v, sem)

    for k in range(_CHUNK // 16):
        ones_v[pl.ds(k * 16, 16)] = zero16 + 1.0

    def zstore(i, carry):
        z_v[pl.ds(i * 16, 16)] = zero16
        return carry

    lax.fori_loop(0, _ZCHUNK // 16, zstore, 0)
    pltpu.sync_copy(z_v, cnt_sh.at[pl.ds(s * 2 * _ZCHUNK, _ZCHUNK)])
    pltpu.sync_copy(z_v, cnt_sh.at[pl.ds((s * 2 + 1) * _ZCHUNK, _ZCHUNK)])
    plsc.subcore_barrier()

    icopy.wait()
    hist_copies = [
        pltpu.async_copy(ones_v, cnt_sh.at[idx_v.at[g]], sem_h, add=True)
        for g in range(_NHC)
    ]
    for g in range(_NGC):
        for k in range(_CHUNK // 16):
            y16 = idx_v[c * _NGC + g, pl.ds(k * 16, 16)]
            idx4_v[g, pl.ds(k * 16, 16)] = y16 >> 2

    ddiag = [(dd + lanes) & 31 for dd in range(_DIM)]

    def transpose_chunk(buf, width):
        def tcol(cb, carry):
            clv = cb * 16 + lanes
            clv32 = clv * 32
            for dd in range(_DIM):
                vals = plsc.load_gather(in_v.at[buf], [ddiag[dd], clv])
                flat = clv32 + ddiag[dd]
                plsc.store_scatter(
                    out_v.at[buf], [flat >> 7, flat & 127], vals)
            return carry
        lax.fori_loop(0, width // 16, tcol, 0)

    def fire_in(k):
        cid = s + 16 * k
        return pltpu.async_copy(
            ct_ref.at[:, pl.ds(cid * _ACH, _ACH)], in_v.at[k % 2], sem_a)

    in_d = [fire_in(0), fire_in(1)]
    out_d = [None, None]
    for k in range(_NF):
        b = k % 2
        in_d[b].wait()
        if out_d[b] is not None:
            out_d[b].wait()
        transpose_chunk(b, _ACH)
        plsc.subcore_barrier()
        cid = s + 16 * k
        out_d[b] = pltpu.async_copy(
            out_v.at[b], my_scr.at[pl.ds(cid * (_ACH * _DIM // 128), 128)],
            sem_o)
        if k + 2 < _NF:
            in_d[b] = fire_in(k + 2)
    for d in out_d:
        d.wait()

    @pl.when(s < 3)
    def _full_epilogue():
        cid = s + 16 * _NF
        pltpu.sync_copy(ct_ref.at[:, pl.ds(cid * _ACH, _ACH)], in_v.at[0])
        transpose_chunk(0, _ACH)

    @pl.when(s == 3)
    def _tail_epilogue():
        pltpu.sync_copy(tail_ref, in_v.at[0].at[:, pl.ds(0, _TAILW)])
        transpose_chunk(0, _TAILW)

    plsc.subcore_barrier()

    @pl.when(s < 3)
    def _full_epilogue_out():
        cid = s + 16 * _NF
        pltpu.sync_copy(out_v.at[0],
                        my_scr.at[pl.ds(cid * (_ACH * _DIM // 128), 128)])

    @pl.when(s == 3)
    def _tail_epilogue_out():
        pltpu.sync_copy(
            out_v.at[0].at[pl.ds(0, _TAILW * _DIM // 128)],
            my_scr.at[pl.ds(195 * (_ACH * _DIM // 128), _TAILW * _DIM // 128)])

    for h in hist_copies:
        h.wait()
    plsc.subcore_barrier()

    cnt_copies = [
        pltpu.async_copy(cnt_sh.at[idx_v.at[c * _NGC + g]],
                         cnt_v.at[pl.ds(g * _CHUNK, _CHUNK)], sem_h)
        for g in range(_NGC)
    ]
    xcopy.wait()

    def fire_c(g):
        return pltpu.async_copy(
            my_scr.at[idx4_v.at[g]], c_v.at[g % 2], sem)

    c_d = [fire_c(0), fire_c(1)]
    acc = zero16
    for g128 in range(_NGC):
        c_d[g128 % 2].wait()
        cnt_copies[g128].wait()

        def group(g, a):
            j0 = g128 * _CHUNK + g * 16
            rows = lanes + g * 16
            y16 = idx_v[c * _NGC + g128, pl.ds(g * 16, 16)]
            ccol0 = (y16 & 3) << 5
            cnt16 = plsc.load_gather(cnt_v, [lanes + j0])
            w16 = 0.5 / (cnt16 + 1.0)
            sq = zero16
            for d in range(_DIM):
                t = (xsT_v[d, pl.ds(j0, 16)]
                     - plsc.load_gather(c_v.at[g128 % 2], [rows, ccol0 + d]))
                sq = sq + t * t
            return a + sq * w16

        acc = lax.fori_loop(0, _CHUNK // 16, group, acc)
        if g128 + 2 < _NGC:
            c_d[g128 % 2] = fire_c(g128 + 2)
    acc_v[...] = acc
    pltpu.sync_copy(acc_v, out_ref.at[pl.ds(wid * 16, 16)])


def kernel(xs, ys, center):
    ys2d = ys.astype(jnp.int32).reshape(_NS * _NHC, _CHUNK)
    xsT = xs.T
    centerT = center.T
    tailT = jnp.pad(center[195 * _ACH:].T, ((0, 0), (0, _TAILW - 160)))
    mesh = plsc.VectorSubcoreMesh(
        core_axis_name="c", subcore_axis_name="s", num_cores=_NC)
    out, _ = pl.kernel(
        _body,
        out_type=(jax.ShapeDtypeStruct((_NW * 16,), jnp.float32),
                  jax.ShapeDtypeStruct((_NC, _SROWS, 128), jnp.float32)),
        mesh=mesh,
        compiler_params=pltpu.CompilerParams(
            needs_layout_passes=False, use_tc_tiling_on_sc=True),
        scratch_types=[
            pltpu.VMEM((_NHC, _CHUNK), jnp.int32),
            pltpu.VMEM((_NGC, _CHUNK), jnp.int32),
            pltpu.VMEM((_DIM, _PER), jnp.float32),
            pltpu.VMEM((2, _DIM, _ACH), jnp.float32),
            pltpu.VMEM((2, 128, 128), jnp.float32),
            pltpu.VMEM((2, _CHUNK, 128), jnp.float32),
            pltpu.VMEM((_PER,), jnp.float32),
            pltpu.VMEM((_CHUNK,), jnp.float32),
            pltpu.VMEM((_ZCHUNK,), jnp.float32),
            pltpu.VMEM((16,), jnp.float32),
            pltpu.VMEM_SHARED((_CNT_PAD,), jnp.float32),
            pltpu.SemaphoreType.DMA,
            pltpu.SemaphoreType.DMA,
            pltpu.SemaphoreType.DMA,
            pltpu.SemaphoreType.DMA,
            pltpu.SemaphoreType.DMA,
        ],
    )(ys2d, xsT, centerT, tailT)
    return jnp.sum(out)


# speedup vs baseline: 4.9325x; 2.3944x over previous
import jax
import jax.numpy as jnp
from jax import lax
from jax.experimental import pallas as pl
from jax.experimental.pallas import tpu as pltpu
from jax.experimental.pallas import tpu_sc as plsc

_CLS = 100000
_DIM = 32
_BATCH = 16384
_NC = 2
_NS = 16
_NW = _NC * _NS
_CHUNK = 128
_NHC = 8
_HIST = _NHC * _CHUNK
_CNT_PAD = 100096
_ZCHUNK = _CNT_PAD // _NS // 2
_Q = 4096


def _body(ys_ref, xsT_ref, ct_ref, out_ref,
          idx_v, crow_v, xrow_v, ysq_v, wq_v, cnt_v, w1k_v, ones_v, z_v,
          acc_v, cnt_sh, w_sh, sem, sem_i, sem_h, sem_x):
    c = lax.axis_index("c")
    s = lax.axis_index("s")
    wid = s * _NC + c
    d = c * _NS + s
    lanes = lax.iota(jnp.int32, 16)
    zero16 = jnp.zeros((16,), jnp.float32)

    ccopy = pltpu.async_copy(ct_ref.at[d], crow_v, sem_x)
    icopy = pltpu.async_copy(ys_ref.at[pl.ds(s * _NHC, _NHC)], idx_v, sem_i)

    for k in range(_CHUNK // 16):
        ones_v[pl.ds(k * 16, 16)] = zero16 + 1.0

    def zstore(i, carry):
        z_v[pl.ds(i * 16, 16)] = zero16
        return carry

    lax.fori_loop(0, _ZCHUNK // 16, zstore, 0)
    pltpu.sync_copy(z_v, cnt_sh.at[pl.ds(s * 2 * _ZCHUNK, _ZCHUNK)])
    pltpu.sync_copy(z_v, cnt_sh.at[pl.ds((s * 2 + 1) * _ZCHUNK, _ZCHUNK)])
    plsc.subcore_barrier()

    icopy.wait()
    hist_copies = [
        pltpu.async_copy(ones_v, cnt_sh.at[idx_v.at[g]], sem_h, add=True)
        for g in range(_NHC)
    ]
    for h in hist_copies:
        h.wait()
    plsc.subcore_barrier()

    cnt_copies = [
        pltpu.async_copy(cnt_sh.at[idx_v.at[g]],
                         cnt_v.at[pl.ds(g * _CHUNK, _CHUNK)], sem_h)
        for g in range(_NHC)
    ]
    for cc in cnt_copies:
        cc.wait()

    def wstore(i, carry):
        cnt16 = cnt_v[pl.ds(i * 16, 16)]
        w1k_v[pl.ds(i * 16, 16)] = 0.5 / (cnt16 + 1.0)
        return carry

    lax.fori_loop(0, _HIST // 16, wstore, 0)
    pltpu.sync_copy(w1k_v, w_sh.at[pl.ds(s * _HIST, _HIST)])
    plsc.subcore_barrier()

    ccopy.wait()
    acc = zero16
    for q in range(_BATCH // _Q):
        pltpu.sync_copy(xsT_ref.at[d, pl.ds(q * _Q, _Q)], xrow_v)
        pltpu.sync_copy(ys_ref.at[pl.ds(q * (_Q // _CHUNK), _Q // _CHUNK)],
                        ysq_v)
        pltpu.sync_copy(w_sh.at[pl.ds(q * _Q, _Q)], wq_v)

        def group(g, a):
            y16 = ysq_v[g >> 3, pl.ds((g & 7) * 16, 16)]
            cv = plsc.load_gather(crow_v, [y16])
            xv = xrow_v[pl.ds(g * 16, 16)]
            w16 = wq_v[pl.ds(g * 16, 16)]
            t = xv - cv
            return a + w16 * t * t

        acc = lax.fori_loop(0, _Q // 16, group, acc)
    acc_v[...] = acc
    pltpu.sync_copy(acc_v, out_ref.at[pl.ds(wid * 16, 16)])


def kernel(xs, ys, center):
    ys2d = ys.astype(jnp.int32).reshape(_BATCH // _CHUNK, _CHUNK)
    xsT = xs.T
    centerT = center.T
    mesh = plsc.VectorSubcoreMesh(
        core_axis_name="c", subcore_axis_name="s", num_cores=_NC)
    out = pl.kernel(
        _body,
        out_type=jax.ShapeDtypeStruct((_NW * 16,), jnp.float32),
        mesh=mesh,
        compiler_params=pltpu.CompilerParams(
            needs_layout_passes=False, use_tc_tiling_on_sc=True),
        scratch_types=[
            pltpu.VMEM((_NHC, _CHUNK), jnp.int32),
            pltpu.VMEM((_CLS,), jnp.float32),
            pltpu.VMEM((_Q,), jnp.float32),
            pltpu.VMEM((_Q // _CHUNK, _CHUNK), jnp.int32),
            pltpu.VMEM((_Q,), jnp.float32),
            pltpu.VMEM((_HIST,), jnp.float32),
            pltpu.VMEM((_HIST,), jnp.float32),
            pltpu.VMEM((_CHUNK,), jnp.float32),
            pltpu.VMEM((_ZCHUNK,), jnp.float32),
            pltpu.VMEM((16,), jnp.float32),
            pltpu.VMEM_SHARED((_CNT_PAD,), jnp.float32),
            pltpu.VMEM_SHARED((_BATCH,), jnp.float32),
            pltpu.SemaphoreType.DMA,
            pltpu.SemaphoreType.DMA,
            pltpu.SemaphoreType.DMA,
            pltpu.SemaphoreType.DMA,
        ],
    )(ys2d, xsT, centerT)
    return jnp.sum(out)


# speedup vs baseline: 5.4202x; 1.0989x over previous
import jax
import jax.numpy as jnp
from jax import lax
from jax.experimental import pallas as pl
from jax.experimental.pallas import tpu as pltpu
from jax.experimental.pallas import tpu_sc as plsc

_CLS = 100000
_DIM = 32
_BATCH = 16384
_NC = 2
_NS = 16
_NW = _NC * _NS
_CHUNK = 128
_NHC = 8
_HIST = _NHC * _CHUNK
_CNT_PAD = 100096
_ZCHUNK = _CNT_PAD // _NS // 2
_Q = 4096


def _body(ys_ref, xsT_ref, ct_ref, out_ref,
          idx_v, crow_v, xrow_v, ysq_v, wq_v, cnt_v, w1k_v, ones_v, z_v,
          acc_v, cnt_sh, w_sh, sem, sem_i, sem_h, sem_x):
    c = lax.axis_index("c")
    s = lax.axis_index("s")
    wid = s * _NC + c
    d = c * _NS + s
    lanes = lax.iota(jnp.int32, 16)
    zero16 = jnp.zeros((16,), jnp.float32)

    ccopy = pltpu.async_copy(ct_ref.at[d], crow_v, sem_x)
    icopy = pltpu.async_copy(ys_ref.at[pl.ds(s * _NHC, _NHC)], idx_v, sem_i)

    for k in range(_CHUNK // 16):
        ones_v[pl.ds(k * 16, 16)] = zero16 + 1.0

    def zstore(i, carry):
        z_v[pl.ds(i * 16, 16)] = zero16
        return carry

    lax.fori_loop(0, _ZCHUNK // 16, zstore, 0)
    pltpu.sync_copy(z_v, cnt_sh.at[pl.ds(s * 2 * _ZCHUNK, _ZCHUNK)])
    pltpu.sync_copy(z_v, cnt_sh.at[pl.ds((s * 2 + 1) * _ZCHUNK, _ZCHUNK)])
    plsc.subcore_barrier()

    icopy.wait()
    hist_copies = [
        pltpu.async_copy(ones_v, cnt_sh.at[idx_v.at[g]], sem_h, add=True)
        for g in range(_NHC)
    ]
    for h in hist_copies:
        h.wait()
    plsc.subcore_barrier()

    cnt_copies = [
        pltpu.async_copy(cnt_sh.at[idx_v.at[g]],
                         cnt_v.at[pl.ds(g * _CHUNK, _CHUNK)], sem_h)
        for g in range(_NHC)
    ]
    for cc in cnt_copies:
        cc.wait()

    def wstore(i, carry):
        cnt16 = cnt_v[pl.ds(i * 16, 16)]
        w1k_v[pl.ds(i * 16, 16)] = 0.5 / (cnt16 + 1.0)
        return carry

    lax.fori_loop(0, _HIST // 16, wstore, 0)
    pltpu.sync_copy(w1k_v, w_sh.at[pl.ds(s * _HIST, _HIST)])
    plsc.subcore_barrier()

    ccopy.wait()
    acc = zero16
    for q in range(_BATCH // _Q):
        stage = [
            pltpu.async_copy(xsT_ref.at[d, pl.ds(q * _Q, _Q)], xrow_v, sem_x),
            pltpu.async_copy(
                ys_ref.at[pl.ds(q * (_Q // _CHUNK), _Q // _CHUNK)],
                ysq_v, sem_i),
            pltpu.async_copy(w_sh.at[pl.ds(q * _Q, _Q)], wq_v, sem_h),
        ]
        for sc in stage:
            sc.wait()

        def group(g, a):
            y16 = ysq_v[g >> 3, pl.ds((g & 7) * 16, 16)]
            cv = plsc.load_gather(crow_v, [y16])
            xv = xrow_v[pl.ds(g * 16, 16)]
            w16 = wq_v[pl.ds(g * 16, 16)]
            t = xv - cv
            return a + w16 * t * t

        acc = lax.fori_loop(0, _Q // 16, group, acc)
    acc_v[...] = acc
    pltpu.sync_copy(acc_v, out_ref.at[pl.ds(wid * 16, 16)])


def kernel(xs, ys, center):
    ys2d = ys.astype(jnp.int32).reshape(_BATCH // _CHUNK, _CHUNK)
    xsT = xs.T
    centerT = center.T
    mesh = plsc.VectorSubcoreMesh(
        core_axis_name="c", subcore_axis_name="s", num_cores=_NC)
    out = pl.kernel(
        _body,
        out_type=jax.ShapeDtypeStruct((_NW * 16,), jnp.float32),
        mesh=mesh,
        compiler_params=pltpu.CompilerParams(
            needs_layout_passes=False, use_tc_tiling_on_sc=True),
        scratch_types=[
            pltpu.VMEM((_NHC, _CHUNK), jnp.int32),
            pltpu.VMEM((_CLS,), jnp.float32),
            pltpu.VMEM((_Q,), jnp.float32),
            pltpu.VMEM((_Q // _CHUNK, _CHUNK), jnp.int32),
            pltpu.VMEM((_Q,), jnp.float32),
            pltpu.VMEM((_HIST,), jnp.float32),
            pltpu.VMEM((_HIST,), jnp.float32),
            pltpu.VMEM((_CHUNK,), jnp.float32),
            pltpu.VMEM((_ZCHUNK,), jnp.float32),
            pltpu.VMEM((16,), jnp.float32),
            pltpu.VMEM_SHARED((_CNT_PAD,), jnp.float32),
            pltpu.VMEM_SHARED((_BATCH,), jnp.float32),
            pltpu.SemaphoreType.DMA,
            pltpu.SemaphoreType.DMA,
            pltpu.SemaphoreType.DMA,
            pltpu.SemaphoreType.DMA,
        ],
    )(ys2d, xsT, centerT)
    return jnp.sum(out)
